# baseline (speedup 1.0000x reference)
import functools

import jax
import jax.numpy as jnp
from jax import lax
from jax.experimental import pallas as pl
from jax.experimental.pallas import tpu as pltpu
from jax.experimental.pallas import tpu_sc as plsc

_NUM_CORES = 2
_NUM_SUBCORES = 16
_NUM_WORKERS = _NUM_CORES * _NUM_SUBCORES
_CHUNK = 128
_NBUF = 2


@functools.cache
def _build_gather(n_chunks: int, vocab: int, dim: int):
    rows_total = _NUM_WORKERS * n_chunks * _CHUNK
    mesh = plsc.VectorSubcoreMesh(core_axis_name="c", subcore_axis_name="s")

    def body(idx_hbm, table_hbm, out_hbm, idx_v, rows0, rows1, sem0, sem1):
        rows = (rows0, rows1)
        sems = (sem0, sem1)
        wid = lax.axis_index("s") * _NUM_CORES + lax.axis_index("c")
        base = wid * (n_chunks * _CHUNK)

        pltpu.sync_copy(idx_hbm.at[wid], idx_v)

        for b in range(_NBUF):
            pltpu.async_copy(table_hbm.at[idx_v.at[b]], rows[b], sems[b])

        @pl.loop(0, n_chunks - _NBUF, step=_NBUF)
        def _steady(g):
            for b in range(_NBUF):
                j = g + b
                pltpu.make_async_copy(
                    table_hbm.at[idx_v.at[j]], rows[b], sems[b]
                ).wait()
                pltpu.sync_copy(
                    rows[b], out_hbm.at[pl.ds(base + j * _CHUNK, _CHUNK)]
                )
                pltpu.async_copy(
                    table_hbm.at[idx_v.at[j + _NBUF]], rows[b], sems[b]
                )

        for b in range(_NBUF):
            j = n_chunks - _NBUF + b
            pltpu.make_async_copy(
                table_hbm.at[idx_v.at[j]], rows[b], sems[b]
            ).wait()
            pltpu.sync_copy(
                rows[b], out_hbm.at[pl.ds(base + j * _CHUNK, _CHUNK)]
            )

    return pl.kernel(
        body,
        out_type=jax.ShapeDtypeStruct((rows_total, dim), jnp.float32),
        mesh=mesh,
        scratch_types=[
            pltpu.VMEM((n_chunks, _CHUNK), jnp.int32),
            pltpu.VMEM((_CHUNK, dim), jnp.float32),
            pltpu.VMEM((_CHUNK, dim), jnp.float32),
            pltpu.SemaphoreType.DMA,
            pltpu.SemaphoreType.DMA,
        ],
        compiler_params=pltpu.CompilerParams(use_tc_tiling_on_sc=False),
    )


def kernel(inputs, embedding):
    batch, seq = inputs.shape
    vocab, dim = embedding.shape
    total = batch * seq
    n_chunks = total // (_NUM_WORKERS * _CHUNK)
    assert total == _NUM_WORKERS * _CHUNK * n_chunks, (batch, seq)
    idx = inputs.reshape(_NUM_WORKERS, n_chunks, _CHUNK).astype(jnp.int32)
    out = _build_gather(n_chunks, vocab, dim)(idx, embedding)
    return out.reshape(batch, seq, dim)


# speedup vs baseline: 1.0139x; 1.0139x over previous
import functools

import jax
import jax.numpy as jnp
from jax import lax
from jax.experimental import pallas as pl
from jax.experimental.pallas import tpu as pltpu
from jax.experimental.pallas import tpu_sc as plsc

_NUM_CORES = 2
_NUM_SUBCORES = 16
_NUM_WORKERS = _NUM_CORES * _NUM_SUBCORES
_CHUNK = 128
_NBUF = 4


@functools.cache
def _build_gather(n_chunks: int, vocab: int, dim: int):
    rows_total = _NUM_WORKERS * n_chunks * _CHUNK
    mesh = plsc.VectorSubcoreMesh(core_axis_name="c", subcore_axis_name="s")

    def body(idx_hbm, table_hbm, out_hbm, idx_v, *scratch):
        rows = scratch[:_NBUF]
        sems = scratch[_NBUF:]
        wid = lax.axis_index("s") * _NUM_CORES + lax.axis_index("c")
        base = wid * (n_chunks * _CHUNK)

        pltpu.sync_copy(idx_hbm.at[wid], idx_v)

        for b in range(_NBUF):
            pltpu.async_copy(table_hbm.at[idx_v.at[b]], rows[b], sems[b])

        @pl.loop(0, n_chunks - _NBUF, step=_NBUF)
        def _steady(g):
            for b in range(_NBUF):
                j = g + b
                pltpu.make_async_copy(
                    table_hbm.at[idx_v.at[j]], rows[b], sems[b]
                ).wait()
                pltpu.sync_copy(
                    rows[b], out_hbm.at[pl.ds(base + j * _CHUNK, _CHUNK)]
                )
                pltpu.async_copy(
                    table_hbm.at[idx_v.at[j + _NBUF]], rows[b], sems[b]
                )

        for b in range(_NBUF):
            j = n_chunks - _NBUF + b
            pltpu.make_async_copy(
                table_hbm.at[idx_v.at[j]], rows[b], sems[b]
            ).wait()
            pltpu.sync_copy(
                rows[b], out_hbm.at[pl.ds(base + j * _CHUNK, _CHUNK)]
            )

    return pl.kernel(
        body,
        out_type=jax.ShapeDtypeStruct((rows_total, dim), jnp.float32),
        mesh=mesh,
        scratch_types=(
            [pltpu.VMEM((n_chunks, _CHUNK), jnp.int32)]
            + [pltpu.VMEM((_CHUNK, dim), jnp.float32) for _ in range(_NBUF)]
            + [pltpu.SemaphoreType.DMA for _ in range(_NBUF)]
        ),
        compiler_params=pltpu.CompilerParams(use_tc_tiling_on_sc=False),
    )


def kernel(inputs, embedding):
    batch, seq = inputs.shape
    vocab, dim = embedding.shape
    total = batch * seq
    n_chunks = total // (_NUM_WORKERS * _CHUNK)
    assert total == _NUM_WORKERS * _CHUNK * n_chunks, (batch, seq)
    idx = inputs.reshape(_NUM_WORKERS, n_chunks, _CHUNK).astype(jnp.int32)
    out = _build_gather(n_chunks, vocab, dim)(idx, embedding)
    return out.reshape(batch, seq, dim)
